# baseline (speedup 1.0000x reference)
import functools
import jax
import jax.numpy as jnp
from jax import lax
from jax.experimental import pallas as pl
from jax.experimental.pallas import tpu as pltpu
from jax.experimental.pallas import tpu_sc as plsc

N = 10000
E = 320000
D = 128
B = 100
L = 100
SMI = 300

NC = 2
NS = 16
NW = NC * NS
CH = 120
NRB = 3
NIB = 6
CPT = 84
LAG = 2
AHD = 4
E_P = NW * CPT * CH
N_ACC = N + 8
SLC = 624
SLC_LAST = N - SLC * (NS - 1)


def _seg_sum_body(h_hbm, src_hbm, dst_hbm, zeros_hbm, out_hbm,
                  acc_sh, idx_v, rows_v, *sems):
    isem = sems[0:NIB]
    rsem = sems[NIB:NIB + NRB]
    asem = sems[NIB + NRB:NIB + 2 * NRB]
    c = lax.axis_index("c")
    s = lax.axis_index("s")
    wid = s * NC + c
    off = pl.multiple_of(s * SLC, 8)

    @pl.when(s < NS - 1)
    def _():
        pltpu.sync_copy(zeros_hbm.at[pl.ds(off, SLC)],
                        acc_sh.at[pl.ds(off, SLC)])

    @pl.when(s == NS - 1)
    def _():
        pltpu.sync_copy(zeros_hbm.at[pl.ds(off, SLC_LAST + 8)],
                        acc_sh.at[pl.ds(off, SLC_LAST + 8)])

    plsc.subcore_barrier()

    def idx_start(j, ib):
        base = pl.multiple_of((wid + j * NW) * CH, 8)
        pltpu.async_copy(src_hbm.at[pl.ds(base, CH)], idx_v.at[ib, 0], isem[ib])
        pltpu.async_copy(dst_hbm.at[pl.ds(base, CH)], idx_v.at[ib, 1], isem[ib])

    def idx_wait(ib):
        pltpu.make_async_copy(src_hbm.at[pl.ds(0, CH)], idx_v.at[ib, 0],
                              isem[ib]).wait()
        pltpu.make_async_copy(dst_hbm.at[pl.ds(0, CH)], idx_v.at[ib, 1],
                              isem[ib]).wait()

    def gather_start(ib, rb):
        pltpu.async_copy(h_hbm.at[idx_v.at[ib, 0]], rows_v.at[rb], rsem[rb])

    def rows_wait(rb):
        pltpu.make_async_copy(h_hbm.at[pl.ds(0, CH)], rows_v.at[rb],
                              rsem[rb]).wait()

    def scatter_start(ib, rb):
        pltpu.async_copy(rows_v.at[rb], acc_sh.at[idx_v.at[ib, 1]], asem[rb],
                         add=True)

    def add_wait(rb):
        pltpu.make_async_copy(rows_v.at[rb], acc_sh.at[pl.ds(0, CH)],
                              asem[rb]).wait()

    for j in range(AHD):
        idx_start(j, j)
    idx_wait(0)
    gather_start(0, 0)

    UN = 6

    def outer(o, carry):
        for u in range(UN):
            i = o * UN + u

            @pl.when(i >= LAG)
            def _():
                add_wait((u - LAG) % NRB)

            @pl.when(i + AHD < CPT)
            def _():
                idx_start(i + AHD, (u + AHD) % NIB)

            @pl.when(i + 1 < CPT)
            def _():
                idx_wait((u + 1) % NIB)
                gather_start((u + 1) % NIB, (u + 1) % NRB)

            rows_wait(u % NRB)
            scatter_start(u % NIB, u % NRB)
        return carry

    lax.fori_loop(0, CPT // UN, outer, 0)
    for k in range(LAG):
        add_wait((CPT - LAG + k) % NRB)
    plsc.subcore_barrier()

    @pl.when(s < NS - 1)
    def _():
        pltpu.sync_copy(acc_sh.at[pl.ds(off, SLC)],
                        out_hbm.at[c, pl.ds(off, SLC)])

    @pl.when(s == NS - 1)
    def _():
        pltpu.sync_copy(acc_sh.at[pl.ds(off, SLC_LAST)],
                        out_hbm.at[c, pl.ds(off, SLC_LAST)])


_seg_sum = pl.kernel(
    _seg_sum_body,
    out_type=jax.ShapeDtypeStruct((NC, N, D), jnp.float32),
    mesh=plsc.VectorSubcoreMesh(core_axis_name="c", subcore_axis_name="s",
                                num_cores=NC, num_subcores=NS),
    scratch_types=[
        pltpu.VMEM_SHARED((N_ACC, D), jnp.float32),
        pltpu.VMEM((NIB, 2, CH), jnp.int32),
        pltpu.VMEM((NRB, CH, D), jnp.float32),
    ] + [pltpu.SemaphoreType.DMA] * (NIB + 2 * NRB),
)


def _init_body(x_ref, w_ref, o_ref):
    o_ref[...] = jnp.dot(x_ref[...], w_ref[...], preferred_element_type=jnp.float32)


def _minit_body(sx_ref, w_ref, o_ref):
    o_ref[...] = jnp.dot(sx_ref[0] + sx_ref[1], w_ref[...],
                         preferred_element_type=jnp.float32)


def _layer_m_body(m_ref, h_ref, wc_ref, bc_ref, wr_ref, br_ref, g_ref, be_ref, o_ref):
    conv = jnp.maximum(jnp.dot(m_ref[...], wc_ref[...], preferred_element_type=jnp.float32) + bc_ref[...], 0.0)
    res = jnp.dot(h_ref[...], wr_ref[...], preferred_element_type=jnp.float32) + br_ref[...]
    o_ref[...] = (conv + res) * g_ref[...] + be_ref[...]


def _layer_body(m_ref, h_ref, wc_ref, bc_ref, wr_ref, br_ref, g_ref, be_ref, o_ref):
    m = m_ref[0] + m_ref[1]
    h = h_ref[...]
    conv = jnp.maximum(jnp.dot(m, wc_ref[...], preferred_element_type=jnp.float32) + bc_ref[...], 0.0)
    res = jnp.dot(h, wr_ref[...], preferred_element_type=jnp.float32) + br_ref[...]
    o_ref[...] = (conv + res) * g_ref[...] + be_ref[...]


def _smiles_body(x_ref, w0_ref, w1_ref, w2_ref, bconv_ref, gs_ref, bs_ref,
                 wp_ref, bp_ref, wf2_ref, bfus_ref, o_ref):
    xb = x_ref[0]
    y0 = jnp.dot(xb, w0_ref[...], preferred_element_type=jnp.float32)
    y1 = jnp.dot(xb, w1_ref[...], preferred_element_type=jnp.float32)
    y2 = jnp.dot(xb, w2_ref[...], preferred_element_type=jnp.float32)
    zrow = jnp.zeros((1, D), jnp.float32)
    c = y1 + jnp.concatenate([zrow, y0[:-1]], axis=0) \
           + jnp.concatenate([y2[1:], zrow], axis=0)
    z = jnp.maximum((c + bconv_ref[...]) * gs_ref[...] + bs_ref[...], 0.0)
    sf = jnp.max(z, axis=0, keepdims=True)
    s1 = jnp.dot(sf, wp_ref[...], preferred_element_type=jnp.float32) + bp_ref[...]
    o_ref[0] = jnp.dot(s1, wf2_ref[...], preferred_element_type=jnp.float32) + bfus_ref[...]


def _fuse_body(h_ref, s2_ref, wf1_ref, gf_ref, bf_ref, o_ref):
    f = jnp.dot(h_ref[0], wf1_ref[...], preferred_element_type=jnp.float32) + s2_ref[0]
    o_ref[0] = jnp.maximum(f * gf_ref[...] + bf_ref[...], 0.0)


def _row(v):
    return v.reshape(1, D)


def kernel(x, edge_index, smiles_embeddings, W_init, Wc0, bc0, Wr0, br0, g0, be0,
           Wc1, bc1, Wr1, br1, g1, be1, Wconv, bconv, gs, bs, Wproj, bproj,
           Wfus, bfus, gf, bf):
    zeros_nd = jnp.zeros((N_ACC, D), jnp.float32)
    src_p = jnp.concatenate([edge_index[0], jnp.zeros((E_P - E,), jnp.int32)])
    dst_p = jnp.concatenate([edge_index[1], jnp.full((E_P - E,), N, jnp.int32)])

    sx = _seg_sum(x, src_p, dst_p, zeros_nd)
    h = pl.pallas_call(
        _init_body,
        out_shape=jax.ShapeDtypeStruct((N, D), jnp.float32),
    )(x, W_init)
    m0 = pl.pallas_call(
        _minit_body,
        out_shape=jax.ShapeDtypeStruct((N, D), jnp.float32),
    )(sx, W_init)
    h = pl.pallas_call(
        _layer_m_body,
        out_shape=jax.ShapeDtypeStruct((N, D), jnp.float32),
    )(m0, h, Wc0, _row(bc0), Wr0, _row(br0), _row(g0), _row(be0))

    m1 = _seg_sum(h, src_p, dst_p, zeros_nd)
    h = pl.pallas_call(
        _layer_body,
        out_shape=jax.ShapeDtypeStruct((N, D), jnp.float32),
    )(m1, h, Wc1, _row(bc1), Wr1, _row(br1), _row(g1), _row(be1))

    Wt = jnp.transpose(Wconv, (1, 0, 2))
    W0, W1, W2 = Wt[:, :, 0], Wt[:, :, 1], Wt[:, :, 2]
    Wf1, Wf2 = Wfus[:D], Wfus[D:]
    s2 = pl.pallas_call(
        _smiles_body,
        grid=(B,),
        in_specs=[
            pl.BlockSpec((1, L, SMI), lambda b: (b, 0, 0)),
            pl.BlockSpec((SMI, D), lambda b: (0, 0)),
            pl.BlockSpec((SMI, D), lambda b: (0, 0)),
            pl.BlockSpec((SMI, D), lambda b: (0, 0)),
            pl.BlockSpec((1, D), lambda b: (0, 0)),
            pl.BlockSpec((1, D), lambda b: (0, 0)),
            pl.BlockSpec((1, D), lambda b: (0, 0)),
            pl.BlockSpec((D, D), lambda b: (0, 0)),
            pl.BlockSpec((1, D), lambda b: (0, 0)),
            pl.BlockSpec((D, D), lambda b: (0, 0)),
            pl.BlockSpec((1, D), lambda b: (0, 0)),
        ],
        out_specs=pl.BlockSpec((1, 1, D), lambda b: (b, 0, 0)),
        out_shape=jax.ShapeDtypeStruct((B, 1, D), jnp.float32),
    )(smiles_embeddings, W0, W1, W2, _row(bconv), _row(gs), _row(bs),
      Wproj, _row(bproj), Wf2, _row(bfus))

    fused = pl.pallas_call(
        _fuse_body,
        grid=(B,),
        in_specs=[
            pl.BlockSpec((1, L, D), lambda b: (b, 0, 0)),
            pl.BlockSpec((1, 1, D), lambda b: (b, 0, 0)),
            pl.BlockSpec((D, D), lambda b: (0, 0)),
            pl.BlockSpec((1, D), lambda b: (0, 0)),
            pl.BlockSpec((1, D), lambda b: (0, 0)),
        ],
        out_specs=pl.BlockSpec((1, L, D), lambda b: (b, 0, 0)),
        out_shape=jax.ShapeDtypeStruct((B, L, D), jnp.float32),
    )(h.reshape(B, L, D), s2, Wf1, _row(gf), _row(bf))

    return fused


# speedup vs baseline: 1.1337x; 1.1337x over previous
import functools
import jax
import jax.numpy as jnp
from jax import lax
from jax.experimental import pallas as pl
from jax.experimental.pallas import tpu as pltpu
from jax.experimental.pallas import tpu_sc as plsc

N = 10000
E = 320000
D = 128
B = 100
L = 100
SMI = 300

NC = 2
NS = 16
NW = NC * NS
CH = 120
NRB = 3
NIB = 6
CPT = 84
LAG = 2
AHD = 4
E_P = NW * CPT * CH
N_ACC = N + 8
SLC = 624
SLC_LAST = N - SLC * (NS - 1)


def _seg_sum_body(h_hbm, src_hbm, dst_hbm, zeros_hbm, out_hbm,
                  acc_sh, idx_v, rows_v, *sems):
    isem = sems[0:NIB]
    rsem = sems[NIB:NIB + NRB]
    asem = sems[NIB + NRB:NIB + 2 * NRB]
    c = lax.axis_index("c")
    s = lax.axis_index("s")
    wid = s * NC + c
    off = pl.multiple_of(s * SLC, 8)

    @pl.when(s < NS - 1)
    def _():
        pltpu.sync_copy(zeros_hbm.at[pl.ds(off, SLC)],
                        acc_sh.at[pl.ds(off, SLC)])

    @pl.when(s == NS - 1)
    def _():
        pltpu.sync_copy(zeros_hbm.at[pl.ds(off, SLC_LAST + 8)],
                        acc_sh.at[pl.ds(off, SLC_LAST + 8)])

    plsc.subcore_barrier()

    def idx_start(j, ib):
        base = pl.multiple_of((wid + j * NW) * CH, 8)
        pltpu.async_copy(src_hbm.at[pl.ds(base, CH)], idx_v.at[ib, 0], isem[ib])
        pltpu.async_copy(dst_hbm.at[pl.ds(base, CH)], idx_v.at[ib, 1], isem[ib])

    def idx_wait(ib):
        pltpu.make_async_copy(src_hbm.at[pl.ds(0, CH)], idx_v.at[ib, 0],
                              isem[ib]).wait()
        pltpu.make_async_copy(dst_hbm.at[pl.ds(0, CH)], idx_v.at[ib, 1],
                              isem[ib]).wait()

    def gather_start(ib, rb):
        pltpu.async_copy(h_hbm.at[idx_v.at[ib, 0]], rows_v.at[rb], rsem[rb])

    def rows_wait(rb):
        pltpu.make_async_copy(h_hbm.at[pl.ds(0, CH)], rows_v.at[rb],
                              rsem[rb]).wait()

    def scatter_start(ib, rb):
        pltpu.async_copy(rows_v.at[rb], acc_sh.at[idx_v.at[ib, 1]], asem[rb],
                         add=True)

    def add_wait(rb):
        pltpu.make_async_copy(rows_v.at[rb], acc_sh.at[pl.ds(0, CH)],
                              asem[rb]).wait()

    for j in range(AHD):
        idx_start(j, j)
    idx_wait(0)
    gather_start(0, 0)

    UN = 6

    def outer(o, carry):
        for u in range(UN):
            i = o * UN + u

            @pl.when(i >= LAG)
            def _():
                add_wait((u - LAG) % NRB)

            @pl.when(i + AHD < CPT)
            def _():
                idx_start(i + AHD, (u + AHD) % NIB)

            @pl.when(i + 1 < CPT)
            def _():
                idx_wait((u + 1) % NIB)
                gather_start((u + 1) % NIB, (u + 1) % NRB)

            rows_wait(u % NRB)
            scatter_start(u % NIB, u % NRB)
        return carry

    lax.fori_loop(0, CPT // UN, outer, 0)
    for k in range(LAG):
        add_wait((CPT - LAG + k) % NRB)
    plsc.subcore_barrier()

    @pl.when(s < NS - 1)
    def _():
        pltpu.sync_copy(acc_sh.at[pl.ds(off, SLC)],
                        out_hbm.at[c, pl.ds(off, SLC)])

    @pl.when(s == NS - 1)
    def _():
        pltpu.sync_copy(acc_sh.at[pl.ds(off, SLC_LAST)],
                        out_hbm.at[c, pl.ds(off, SLC_LAST)])


@functools.lru_cache(maxsize=1)
def _make_seg_sum():
    return pl.kernel(
        _seg_sum_body,
        out_type=jax.ShapeDtypeStruct((NC, N, D), jnp.float32),
        mesh=plsc.VectorSubcoreMesh(core_axis_name="c", subcore_axis_name="s",
                                    num_cores=NC, num_subcores=NS),
        scratch_types=[
            pltpu.VMEM_SHARED((N_ACC, D), jnp.float32),
            pltpu.VMEM((NIB, 2, CH), jnp.int32),
            pltpu.VMEM((NRB, CH, D), jnp.float32),
        ] + [pltpu.SemaphoreType.DMA] * (NIB + 2 * NRB),
    )


def _seg_sum(h, src_p, dst_p, zeros_nd):
    return _make_seg_sum()(h, src_p, dst_p, zeros_nd)


def _init_body(x_ref, w_ref, o_ref):
    o_ref[...] = jnp.dot(x_ref[...], w_ref[...], preferred_element_type=jnp.float32)


def _minit_body(sx_ref, w_ref, o_ref):
    o_ref[...] = jnp.dot(sx_ref[0] + sx_ref[1], w_ref[...],
                         preferred_element_type=jnp.float32)


def _layer_m_body(m_ref, h_ref, wc_ref, bc_ref, wr_ref, br_ref, g_ref, be_ref, o_ref):
    conv = jnp.maximum(jnp.dot(m_ref[...], wc_ref[...], preferred_element_type=jnp.float32) + bc_ref[...], 0.0)
    res = jnp.dot(h_ref[...], wr_ref[...], preferred_element_type=jnp.float32) + br_ref[...]
    o_ref[...] = (conv + res) * g_ref[...] + be_ref[...]


def _layer_body(m_ref, h_ref, wc_ref, bc_ref, wr_ref, br_ref, g_ref, be_ref, o_ref):
    m = m_ref[0] + m_ref[1]
    h = h_ref[...]
    conv = jnp.maximum(jnp.dot(m, wc_ref[...], preferred_element_type=jnp.float32) + bc_ref[...], 0.0)
    res = jnp.dot(h, wr_ref[...], preferred_element_type=jnp.float32) + br_ref[...]
    o_ref[...] = (conv + res) * g_ref[...] + be_ref[...]


def _smiles_body(x_ref, w0_ref, w1_ref, w2_ref, bconv_ref, gs_ref, bs_ref,
                 wp_ref, bp_ref, wf2_ref, bfus_ref, o_ref):
    xb = x_ref[0]
    y0 = jnp.dot(xb, w0_ref[...], preferred_element_type=jnp.float32)
    y1 = jnp.dot(xb, w1_ref[...], preferred_element_type=jnp.float32)
    y2 = jnp.dot(xb, w2_ref[...], preferred_element_type=jnp.float32)
    zrow = jnp.zeros((1, D), jnp.float32)
    c = y1 + jnp.concatenate([zrow, y0[:-1]], axis=0) \
           + jnp.concatenate([y2[1:], zrow], axis=0)
    z = jnp.maximum((c + bconv_ref[...]) * gs_ref[...] + bs_ref[...], 0.0)
    sf = jnp.max(z, axis=0, keepdims=True)
    s1 = jnp.dot(sf, wp_ref[...], preferred_element_type=jnp.float32) + bp_ref[...]
    o_ref[0] = jnp.dot(s1, wf2_ref[...], preferred_element_type=jnp.float32) + bfus_ref[...]


def _fuse_body(h_ref, s2_ref, wf1_ref, gf_ref, bf_ref, o_ref):
    f = jnp.dot(h_ref[0], wf1_ref[...], preferred_element_type=jnp.float32) + s2_ref[0]
    o_ref[0] = jnp.maximum(f * gf_ref[...] + bf_ref[...], 0.0)


def _initsmi_body(x_ref, xs_ref, w_ref, w0_ref, w1_ref, w2_ref, bconv_ref,
                  gs_ref, bs_ref, wp_ref, bp_ref, wf2_ref, bfus_ref,
                  h0_ref, s2_ref):
    h0_ref[...] = jnp.dot(x_ref[...], w_ref[...], preferred_element_type=jnp.float32)
    zrow = jnp.zeros((1, D), jnp.float32)
    for t in range(2):
        xb = xs_ref[t]
        y0 = jnp.dot(xb, w0_ref[...], preferred_element_type=jnp.float32)
        y1 = jnp.dot(xb, w1_ref[...], preferred_element_type=jnp.float32)
        y2 = jnp.dot(xb, w2_ref[...], preferred_element_type=jnp.float32)
        c = y1 + jnp.concatenate([zrow, y0[:-1]], axis=0) \
               + jnp.concatenate([y2[1:], zrow], axis=0)
        z = jnp.maximum((c + bconv_ref[...]) * gs_ref[...] + bs_ref[...], 0.0)
        sf = jnp.max(z, axis=0, keepdims=True)
        s1 = jnp.dot(sf, wp_ref[...], preferred_element_type=jnp.float32) + bp_ref[...]
        s2_ref[t] = jnp.dot(s1, wf2_ref[...], preferred_element_type=jnp.float32) + bfus_ref[...]


def _mlayer0_body(sx_ref, h0_ref, wi_ref, wc_ref, bc_ref, wr_ref, br_ref,
                  g_ref, be_ref, o_ref):
    m0 = jnp.dot(sx_ref[0] + sx_ref[1], wi_ref[...], preferred_element_type=jnp.float32)
    conv = jnp.maximum(jnp.dot(m0, wc_ref[...], preferred_element_type=jnp.float32) + bc_ref[...], 0.0)
    res = jnp.dot(h0_ref[...], wr_ref[...], preferred_element_type=jnp.float32) + br_ref[...]
    o_ref[...] = (conv + res) * g_ref[...] + be_ref[...]


def _layerfuse_body(m_ref, h_ref, wc_ref, bc_ref, wr_ref, br_ref, g_ref,
                    be_ref, wf1_ref, s2x_ref, gf_ref, bf_ref, o_ref):
    m = m_ref[0] + m_ref[1]
    conv = jnp.maximum(jnp.dot(m, wc_ref[...], preferred_element_type=jnp.float32) + bc_ref[...], 0.0)
    res = jnp.dot(h_ref[...], wr_ref[...], preferred_element_type=jnp.float32) + br_ref[...]
    h2 = (conv + res) * g_ref[...] + be_ref[...]
    f = jnp.dot(h2, wf1_ref[...], preferred_element_type=jnp.float32) + s2x_ref[...]
    o_ref[...] = jnp.maximum(f * gf_ref[...] + bf_ref[...], 0.0)


def _row(v):
    return v.reshape(1, D)


def kernel(x, edge_index, smiles_embeddings, W_init, Wc0, bc0, Wr0, br0, g0, be0,
           Wc1, bc1, Wr1, br1, g1, be1, Wconv, bconv, gs, bs, Wproj, bproj,
           Wfus, bfus, gf, bf):
    zeros_nd = jnp.zeros((N_ACC, D), jnp.float32)
    src_p = jnp.concatenate([edge_index[0], jnp.zeros((E_P - E,), jnp.int32)])
    dst_p = jnp.concatenate([edge_index[1], jnp.full((E_P - E,), N, jnp.int32)])

    Wt = jnp.transpose(Wconv, (1, 0, 2))
    W0, W1, W2 = Wt[:, :, 0], Wt[:, :, 1], Wt[:, :, 2]
    Wf1, Wf2 = Wfus[:D], Wfus[D:]

    sx = _seg_sum(x, src_p, dst_p, zeros_nd)

    wspec = pl.BlockSpec((SMI, D), lambda b: (0, 0))
    vspec = pl.BlockSpec((1, D), lambda b: (0, 0))
    dspec = pl.BlockSpec((D, D), lambda b: (0, 0))
    h0, s2 = pl.pallas_call(
        _initsmi_body,
        grid=(B // 2,),
        in_specs=[
            pl.BlockSpec((2 * L, D), lambda b: (b, 0)),
            pl.BlockSpec((2, L, SMI), lambda b: (b, 0, 0)),
            dspec, wspec, wspec, wspec, vspec, vspec, vspec,
            dspec, vspec, dspec, vspec,
        ],
        out_specs=[
            pl.BlockSpec((2 * L, D), lambda b: (b, 0)),
            pl.BlockSpec((2, 1, D), lambda b: (b, 0, 0)),
        ],
        out_shape=[
            jax.ShapeDtypeStruct((N, D), jnp.float32),
            jax.ShapeDtypeStruct((B, 1, D), jnp.float32),
        ],
    )(x, smiles_embeddings, W_init, W0, W1, W2, _row(bconv), _row(gs),
      _row(bs), Wproj, _row(bproj), Wf2, _row(bfus))

    h1 = pl.pallas_call(
        _mlayer0_body,
        out_shape=jax.ShapeDtypeStruct((N, D), jnp.float32),
    )(sx, h0, W_init, Wc0, _row(bc0), Wr0, _row(br0), _row(g0), _row(be0))

    m1 = _seg_sum(h1, src_p, dst_p, zeros_nd)

    s2x = jnp.broadcast_to(s2, (B, L, D)).reshape(N, D)
    fused = pl.pallas_call(
        _layerfuse_body,
        out_shape=jax.ShapeDtypeStruct((N, D), jnp.float32),
    )(m1, h1, Wc1, _row(bc1), Wr1, _row(br1), _row(g1), _row(be1),
      Wf1, s2x, _row(gf), _row(bf))

    return fused.reshape(B, L, D)


# speedup vs baseline: 1.1343x; 1.0005x over previous
import functools
import jax
import jax.numpy as jnp
from jax import lax
from jax.experimental import pallas as pl
from jax.experimental.pallas import tpu as pltpu
from jax.experimental.pallas import tpu_sc as plsc

N = 10000
E = 320000
D = 128
B = 100
L = 100
SMI = 300

NC = 2
NS = 16
NW = NC * NS
CH = 120
NRB = 3
NIB = 6
CPT = 84
LAG = 2
AHD = 4
E_P = NW * CPT * CH
N_ACC = N + 8
SLC = 624
SLC_LAST = N - SLC * (NS - 1)


def _seg_sum_body(h_hbm, src_hbm, dst_hbm, zeros_hbm, out_hbm,
                  acc_sh, idx_v, rows_v, *sems):
    isem = sems[0:NIB]
    rsem = sems[NIB:NIB + NRB]
    asem = sems[NIB + NRB:NIB + 2 * NRB]
    c = lax.axis_index("c")
    s = lax.axis_index("s")
    wid = s * NC + c
    off = pl.multiple_of(s * SLC, 8)

    @pl.when(s < NS - 1)
    def _():
        pltpu.sync_copy(zeros_hbm.at[pl.ds(off, SLC)],
                        acc_sh.at[pl.ds(off, SLC)])

    @pl.when(s == NS - 1)
    def _():
        pltpu.sync_copy(zeros_hbm.at[pl.ds(off, SLC_LAST + 8)],
                        acc_sh.at[pl.ds(off, SLC_LAST + 8)])

    plsc.subcore_barrier()

    def idx_start(j, ib):
        base = pl.multiple_of((wid + j * NW) * CH, 8)
        pltpu.async_copy(src_hbm.at[pl.ds(base, CH)], idx_v.at[ib, 0], isem[ib])
        pltpu.async_copy(dst_hbm.at[pl.ds(base, CH)], idx_v.at[ib, 1], isem[ib])

    def idx_wait(ib):
        pltpu.make_async_copy(src_hbm.at[pl.ds(0, CH)], idx_v.at[ib, 0],
                              isem[ib]).wait()
        pltpu.make_async_copy(dst_hbm.at[pl.ds(0, CH)], idx_v.at[ib, 1],
                              isem[ib]).wait()

    def gather_start(ib, rb):
        pltpu.async_copy(h_hbm.at[idx_v.at[ib, 0]], rows_v.at[rb], rsem[rb])

    def rows_wait(rb):
        pltpu.make_async_copy(h_hbm.at[pl.ds(0, CH)], rows_v.at[rb],
                              rsem[rb]).wait()

    def scatter_start(ib, rb):
        pltpu.async_copy(rows_v.at[rb], acc_sh.at[idx_v.at[ib, 1]], asem[rb],
                         add=True)

    def add_wait(rb):
        pltpu.make_async_copy(rows_v.at[rb], acc_sh.at[pl.ds(0, CH)],
                              asem[rb]).wait()

    for j in range(AHD):
        idx_start(j, j)
    idx_wait(0)
    gather_start(0, 0)

    UN = 6

    def outer(o, carry):
        for u in range(UN):
            i = o * UN + u

            @pl.when(i >= LAG)
            def _():
                add_wait((u - LAG) % NRB)

            @pl.when(i + AHD < CPT)
            def _():
                idx_start(i + AHD, (u + AHD) % NIB)

            @pl.when(i + 1 < CPT)
            def _():
                idx_wait((u + 1) % NIB)
                gather_start((u + 1) % NIB, (u + 1) % NRB)

            rows_wait(u % NRB)
            scatter_start(u % NIB, u % NRB)
        return carry

    lax.fori_loop(0, CPT // UN, outer, 0)
    for k in range(LAG):
        add_wait((CPT - LAG + k) % NRB)
    plsc.subcore_barrier()

    @pl.when(s < NS - 1)
    def _():
        pltpu.sync_copy(acc_sh.at[pl.ds(off, SLC)],
                        out_hbm.at[c, pl.ds(off, SLC)])

    @pl.when(s == NS - 1)
    def _():
        pltpu.sync_copy(acc_sh.at[pl.ds(off, SLC_LAST)],
                        out_hbm.at[c, pl.ds(off, SLC_LAST)])


@functools.lru_cache(maxsize=1)
def _make_seg_sum():
    return pl.kernel(
        _seg_sum_body,
        out_type=jax.ShapeDtypeStruct((NC, N, D), jnp.float32),
        mesh=plsc.VectorSubcoreMesh(core_axis_name="c", subcore_axis_name="s",
                                    num_cores=NC, num_subcores=NS),
        scratch_types=[
            pltpu.VMEM_SHARED((N_ACC, D), jnp.float32),
            pltpu.VMEM((NIB, 2, CH), jnp.int32),
            pltpu.VMEM((NRB, CH, D), jnp.float32),
        ] + [pltpu.SemaphoreType.DMA] * (NIB + 2 * NRB),
    )


def _seg_sum(h, src_p, dst_p, zeros_nd):
    return _make_seg_sum()(h, src_p, dst_p, zeros_nd)


def _initsmi_body(x_ref, xs_ref, w_ref, w0_ref, w1_ref, w2_ref, bconv_ref,
                  gs_ref, bs_ref, wp_ref, bp_ref, wf2_ref, bfus_ref,
                  h0_ref, s2_ref):
    h0_ref[...] = jnp.dot(x_ref[...], w_ref[...], preferred_element_type=jnp.float32)
    zrow = jnp.zeros((1, D), jnp.float32)
    for t in range(2):
        xb = xs_ref[t]
        y0 = jnp.dot(xb, w0_ref[...], preferred_element_type=jnp.float32)
        y1 = jnp.dot(xb, w1_ref[...], preferred_element_type=jnp.float32)
        y2 = jnp.dot(xb, w2_ref[...], preferred_element_type=jnp.float32)
        c = y1 + jnp.concatenate([zrow, y0[:-1]], axis=0) \
               + jnp.concatenate([y2[1:], zrow], axis=0)
        z = jnp.maximum((c + bconv_ref[...]) * gs_ref[...] + bs_ref[...], 0.0)
        sf = jnp.max(z, axis=0, keepdims=True)
        s1 = jnp.dot(sf, wp_ref[...], preferred_element_type=jnp.float32) + bp_ref[...]
        s2_ref[t] = jnp.dot(s1, wf2_ref[...], preferred_element_type=jnp.float32) + bfus_ref[...]


def _mlayer0_body(sx_ref, h0_ref, wi_ref, wc_ref, bc_ref, wr_ref, br_ref,
                  g_ref, be_ref, o_ref):
    m0 = jnp.dot(sx_ref[0] + sx_ref[1], wi_ref[...], preferred_element_type=jnp.float32)
    conv = jnp.maximum(jnp.dot(m0, wc_ref[...], preferred_element_type=jnp.float32) + bc_ref[...], 0.0)
    res = jnp.dot(h0_ref[...], wr_ref[...], preferred_element_type=jnp.float32) + br_ref[...]
    o_ref[...] = (conv + res) * g_ref[...] + be_ref[...]


def _layerfuse_body(m_ref, h_ref, wc_ref, bc_ref, wr_ref, br_ref, g_ref,
                    be_ref, wf1_ref, s2x_ref, gf_ref, bf_ref, o_ref):
    m = m_ref[0] + m_ref[1]
    conv = jnp.maximum(jnp.dot(m, wc_ref[...], preferred_element_type=jnp.float32) + bc_ref[...], 0.0)
    res = jnp.dot(h_ref[...], wr_ref[...], preferred_element_type=jnp.float32) + br_ref[...]
    h2 = (conv + res) * g_ref[...] + be_ref[...]
    f = jnp.dot(h2, wf1_ref[...], preferred_element_type=jnp.float32) + s2x_ref[...]
    o_ref[...] = jnp.maximum(f * gf_ref[...] + bf_ref[...], 0.0)


def _row(v):
    return v.reshape(1, D)


def kernel(x, edge_index, smiles_embeddings, W_init, Wc0, bc0, Wr0, br0, g0, be0,
           Wc1, bc1, Wr1, br1, g1, be1, Wconv, bconv, gs, bs, Wproj, bproj,
           Wfus, bfus, gf, bf):
    zeros_nd = jnp.zeros((N_ACC, D), jnp.float32)
    src_p = jnp.concatenate([edge_index[0], jnp.zeros((E_P - E,), jnp.int32)])
    dst_p = jnp.concatenate([edge_index[1], jnp.full((E_P - E,), N, jnp.int32)])

    Wt = jnp.transpose(Wconv, (1, 0, 2))
    W0, W1, W2 = Wt[:, :, 0], Wt[:, :, 1], Wt[:, :, 2]
    Wf1, Wf2 = Wfus[:D], Wfus[D:]

    sx = _seg_sum(x, src_p, dst_p, zeros_nd)

    wspec = pl.BlockSpec((SMI, D), lambda b: (0, 0))
    vspec = pl.BlockSpec((1, D), lambda b: (0, 0))
    dspec = pl.BlockSpec((D, D), lambda b: (0, 0))
    h0, s2 = pl.pallas_call(
        _initsmi_body,
        grid=(B // 2,),
        in_specs=[
            pl.BlockSpec((2 * L, D), lambda b: (b, 0)),
            pl.BlockSpec((2, L, SMI), lambda b: (b, 0, 0)),
            dspec, wspec, wspec, wspec, vspec, vspec, vspec,
            dspec, vspec, dspec, vspec,
        ],
        out_specs=[
            pl.BlockSpec((2 * L, D), lambda b: (b, 0)),
            pl.BlockSpec((2, 1, D), lambda b: (b, 0, 0)),
        ],
        out_shape=[
            jax.ShapeDtypeStruct((N, D), jnp.float32),
            jax.ShapeDtypeStruct((B, 1, D), jnp.float32),
        ],
    )(x, smiles_embeddings, W_init, W0, W1, W2, _row(bconv), _row(gs),
      _row(bs), Wproj, _row(bproj), Wf2, _row(bfus))

    h1 = pl.pallas_call(
        _mlayer0_body,
        out_shape=jax.ShapeDtypeStruct((N, D), jnp.float32),
    )(sx, h0, W_init, Wc0, _row(bc0), Wr0, _row(br0), _row(g0), _row(be0))

    m1 = _seg_sum(h1, src_p, dst_p, zeros_nd)

    s2x = jnp.broadcast_to(s2, (B, L, D)).reshape(N, D)
    fused = pl.pallas_call(
        _layerfuse_body,
        out_shape=jax.ShapeDtypeStruct((N, D), jnp.float32),
    )(m1, h1, Wc1, _row(bc1), Wr1, _row(br1), _row(g1), _row(be1),
      Wf1, s2x, _row(gf), _row(bf))

    return fused.reshape(B, L, D)
